# baseline (speedup 1.0000x reference)
import functools

import jax
import jax.numpy as jnp
from jax import lax
from jax.experimental import pallas as pl
from jax.experimental.pallas import tpu as pltpu
from jax.experimental.pallas import tpu_sc as plsc

MARGIN = 0.2
EPS = 1e-12

V, D = 16384, 128
DP = D // 2
B = 32768
NC, NS, L = 2, 16, 16
NW = NC * NS
TW = B // NW
CH = 128
NCHUNK = TW // CH
NRING = 4
IDX_ROWS = B // CH


def _sqrt16(x):
    i = plsc.bitcast(x, jnp.int32)
    y = plsc.bitcast(jnp.int32(0x5F3759DF) - (i >> 1), jnp.float32)
    xh = x * 0.5
    y = y * (1.5 - xh * y * y)
    y = y * (1.5 - xh * y * y)
    y = y * (1.5 - xh * y * y)
    return x * y


def _unpack2(x_i32):
    ab = plsc.bitcast(x_i32, jnp.bfloat16)
    return plsc.unpack(ab, format=plsc.PackFormat.INTERLEAVED)


def _tl_body(emb, aidx, pidx, nidx,
             out_ap, out_an, out_td, out_part,
             aidx_v, pidx_v, nidx_v,
             bufs_flat, ap_v, an_v, loss_v, sems):
    wid = lax.axis_index("s") * NC + lax.axis_index("c")
    base = wid * TW

    pltpu.sync_copy(aidx.at[pl.ds(wid * NCHUNK, NCHUNK)], aidx_v)
    pltpu.sync_copy(pidx.at[pl.ds(wid * NCHUNK, NCHUNK)], pidx_v)
    pltpu.sync_copy(nidx.at[pl.ds(wid * NCHUNK, NCHUNK)], nidx_v)

    iota = lax.iota(jnp.int32, L)
    bufs = tuple(tuple(bufs_flat[s * 3:s * 3 + 3]) + (sems[s],)
                 for s in range(NRING))

    def fire(c):
        a_buf, p_buf, n_buf, sem = bufs[c % NRING]
        return (pltpu.async_copy(emb.at[aidx_v.at[c]], a_buf, sem),
                pltpu.async_copy(emb.at[pidx_v.at[c]], p_buf, sem),
                pltpu.async_copy(emb.at[nidx_v.at[c]], n_buf, sem))

    def chunk_compute(c, loss_acc):
        a_buf, p_buf, n_buf, _ = bufs[c % NRING]

        def group_body(g, acc):
            row = jnp.full((L,), g * L, dtype=jnp.int32) + iota

            def d_body(dd, carry):
                ap0, ap1, an0, an1, col = carry
                for _ in range(8):
                    xa = plsc.load_gather(a_buf, [row, col])
                    xp = plsc.load_gather(p_buf, [row, col])
                    xn = plsc.load_gather(n_buf, [row, col])
                    av0, av1 = _unpack2(xa)
                    pv0, pv1 = _unpack2(xp)
                    nv0, nv1 = _unpack2(xn)
                    dap0 = av0 - pv0 + EPS
                    dap1 = av1 - pv1 + EPS
                    dan0 = av0 - nv0 + EPS
                    dan1 = av1 - nv1 + EPS
                    ap0 = ap0 + dap0 * dap0
                    ap1 = ap1 + dap1 * dap1
                    an0 = an0 + dan0 * dan0
                    an1 = an1 + dan1 * dan1
                    col = (col + 1) & (DP - 1)
                return ap0, ap1, an0, an1, col

            z = jnp.zeros((L,), jnp.float32)
            ap0, ap1, an0, an1, _ = lax.fori_loop(
                0, DP // 8, d_body, (z, z, z, z, iota))
            ap = _sqrt16(ap0 + ap1)
            an = _sqrt16(an0 + an1)
            off = c * CH + g * L
            ap_v[pl.ds(off, L)] = ap
            an_v[pl.ds(off, L)] = an
            return acc + jnp.maximum(ap - an + MARGIN, 0.0)

        return lax.fori_loop(0, CH // L, group_body, loss_acc)

    loss_acc = jnp.zeros((L,), jnp.float32)
    handles = {}
    for c in range(NRING - 1):
        handles[c] = fire(c)
    for c in range(NCHUNK):
        for h in handles.pop(c):
            h.wait()
        nxt = c + NRING - 1
        if nxt < NCHUNK:
            handles[nxt] = fire(nxt)
        loss_acc = chunk_compute(c, loss_acc)

    loss_v[...] = loss_acc
    pltpu.sync_copy(loss_v, out_part.at[wid])
    pltpu.sync_copy(ap_v, out_ap.at[pl.ds(base, TW)])
    pltpu.sync_copy(an_v, out_an.at[pl.ds(base, TW)])
    pltpu.sync_copy(ap_v, out_td.at[pl.ds(base, TW)])
    pltpu.sync_copy(an_v, out_td.at[pl.ds(B + base, TW)])


_tl_kernel = functools.partial(
    pl.kernel,
    mesh=plsc.VectorSubcoreMesh(core_axis_name="c", subcore_axis_name="s"),
    compiler_params=pltpu.CompilerParams(
        needs_layout_passes=False, use_tc_tiling_on_sc=False),
    out_type=[
        jax.ShapeDtypeStruct((B,), jnp.float32),
        jax.ShapeDtypeStruct((B,), jnp.float32),
        jax.ShapeDtypeStruct((2 * B,), jnp.float32),
        jax.ShapeDtypeStruct((NW, L), jnp.float32),
    ],
    scratch_types=[
        pltpu.VMEM((NCHUNK, CH), jnp.int32),
        pltpu.VMEM((NCHUNK, CH), jnp.int32),
        pltpu.VMEM((NCHUNK, CH), jnp.int32),
        [pltpu.VMEM((CH, DP), jnp.int32) for _ in range(3 * NRING)],
        pltpu.VMEM((TW,), jnp.float32),
        pltpu.VMEM((TW,), jnp.float32),
        pltpu.VMEM((L,), jnp.float32),
        [pltpu.SemaphoreType.DMA for _ in range(NRING)],
    ],
)(_tl_body)


def _pack_bf16(embeddings):
    x = lax.bitcast_convert_type(embeddings, jnp.int32).reshape(V, DP, 2)
    lo, hi = x[..., 0], x[..., 1]
    lo_r = (lo + 0x7FFF + ((lo >> 16) & 1)) >> 16
    hi_r = hi + 0x7FFF + ((hi >> 16) & 1)
    return (hi_r & jnp.int32(-65536)) | (lo_r & 0xFFFF)


def kernel(embeddings, target, triplets):
    del target
    emb_pack = _pack_bf16(embeddings)
    aidx = triplets[:, 0].reshape(IDX_ROWS, CH)
    pidx = triplets[:, 1].reshape(IDX_ROWS, CH)
    nidx = triplets[:, 2].reshape(IDX_ROWS, CH)
    out_ap, out_an, out_td, out_part = _tl_kernel(emb_pack, aidx, pidx, nidx)
    loss = jnp.sum(out_part) / B
    tt = jnp.concatenate(
        [jnp.ones((B,), jnp.float32), jnp.zeros((B,), jnp.float32)])
    return loss, out_ap, out_an, out_td, tt


# speedup vs baseline: 1.8634x; 1.8634x over previous
import functools

import jax
import jax.numpy as jnp
from jax import lax
from jax.experimental import pallas as pl
from jax.experimental.pallas import tpu as pltpu
from jax.experimental.pallas import tpu_sc as plsc

MARGIN = 0.2
EPS = 1e-12

V, D = 16384, 128
DP = D // 2
B = 32768
NC, NS, L = 2, 16, 16
NW = NC * NS
TW = B // NW
CH = 128
NCHUNK = TW // CH
NRING = 4
IDX_ROWS = B // CH


def _sqrt16(x):
    i = plsc.bitcast(x, jnp.int32)
    y = plsc.bitcast(jnp.int32(0x5F3759DF) - (i >> 1), jnp.float32)
    xh = x * 0.5
    y = y * (1.5 - xh * y * y)
    y = y * (1.5 - xh * y * y)
    y = y * (1.5 - xh * y * y)
    return x * y


def _unpack2(x_i32):
    ab = plsc.bitcast(x_i32, jnp.bfloat16)
    return plsc.unpack(ab, format=plsc.PackFormat.INTERLEAVED)


def _tl_body(emb, aidx, pidx, nidx,
             out_ap, out_an, out_td, out_part,
             aidx_v, pidx_v, nidx_v,
             bufs_flat, ap_v, an_v, loss_v, sems):
    wid = lax.axis_index("s") * NC + lax.axis_index("c")
    base = wid * TW

    pltpu.sync_copy(aidx.at[pl.ds(wid * NCHUNK, NCHUNK)], aidx_v)
    pltpu.sync_copy(pidx.at[pl.ds(wid * NCHUNK, NCHUNK)], pidx_v)
    pltpu.sync_copy(nidx.at[pl.ds(wid * NCHUNK, NCHUNK)], nidx_v)

    iota = lax.iota(jnp.int32, L)
    bufs = tuple(tuple(bufs_flat[s * 3:s * 3 + 3]) + (sems[s],)
                 for s in range(NRING))

    def fire(c):
        a_buf, p_buf, n_buf, sem = bufs[c % NRING]
        return (pltpu.async_copy(emb.at[aidx_v.at[c]], a_buf, sem),
                pltpu.async_copy(emb.at[pidx_v.at[c]], p_buf, sem),
                pltpu.async_copy(emb.at[nidx_v.at[c]], n_buf, sem))

    def chunk_compute(c, loss_acc):
        a_buf, p_buf, n_buf, _ = bufs[c % NRING]

        def group_body(g, acc):
            row = jnp.full((L,), g * L, dtype=jnp.int32) + iota

            def d_body(dd, carry):
                ap0, ap1, an0, an1, col = carry
                for _ in range(8):
                    xa = plsc.load_gather(a_buf, [row, col])
                    xp = plsc.load_gather(p_buf, [row, col])
                    xn = plsc.load_gather(n_buf, [row, col])
                    av0, av1 = _unpack2(xa)
                    pv0, pv1 = _unpack2(xp)
                    nv0, nv1 = _unpack2(xn)
                    dap0 = av0 - pv0 + EPS
                    dap1 = av1 - pv1 + EPS
                    dan0 = av0 - nv0 + EPS
                    dan1 = av1 - nv1 + EPS
                    ap0 = ap0 + dap0 * dap0
                    ap1 = ap1 + dap1 * dap1
                    an0 = an0 + dan0 * dan0
                    an1 = an1 + dan1 * dan1
                    col = (col + 1) & (DP - 1)
                return ap0, ap1, an0, an1, col

            z = jnp.zeros((L,), jnp.float32)
            ap0, ap1, an0, an1, _ = lax.fori_loop(
                0, DP // 8, d_body, (z, z, z, z, iota))
            ap = _sqrt16(ap0 + ap1)
            an = _sqrt16(an0 + an1)
            off = c * CH + g * L
            ap_v[pl.ds(off, L)] = ap
            an_v[pl.ds(off, L)] = an
            return acc + jnp.maximum(ap - an + MARGIN, 0.0)

        return lax.fori_loop(0, CH // L, group_body, loss_acc)

    loss_acc = jnp.zeros((L,), jnp.float32)
    handles = {}
    for c in range(NRING - 1):
        handles[c] = fire(c)
    for c in range(NCHUNK):
        for h in handles.pop(c):
            h.wait()
        nxt = c + NRING - 1
        if nxt < NCHUNK:
            handles[nxt] = fire(nxt)
        loss_acc = chunk_compute(c, loss_acc)

    loss_v[...] = loss_acc
    pltpu.sync_copy(loss_v, out_part.at[wid])
    pltpu.sync_copy(ap_v, out_ap.at[pl.ds(base, TW)])
    pltpu.sync_copy(an_v, out_an.at[pl.ds(base, TW)])
    pltpu.sync_copy(ap_v, out_td.at[pl.ds(base, TW)])
    pltpu.sync_copy(an_v, out_td.at[pl.ds(B + base, TW)])


_tl_kernel = functools.partial(
    pl.kernel,
    mesh=plsc.VectorSubcoreMesh(core_axis_name="c", subcore_axis_name="s"),
    compiler_params=pltpu.CompilerParams(
        needs_layout_passes=False, use_tc_tiling_on_sc=False),
    out_type=[
        jax.ShapeDtypeStruct((B,), jnp.float32),
        jax.ShapeDtypeStruct((B,), jnp.float32),
        jax.ShapeDtypeStruct((2 * B,), jnp.float32),
        jax.ShapeDtypeStruct((NW, L), jnp.float32),
    ],
    scratch_types=[
        pltpu.VMEM((NCHUNK, CH), jnp.int32),
        pltpu.VMEM((NCHUNK, CH), jnp.int32),
        pltpu.VMEM((NCHUNK, CH), jnp.int32),
        [pltpu.VMEM((CH, DP), jnp.int32) for _ in range(3 * NRING)],
        pltpu.VMEM((TW,), jnp.float32),
        pltpu.VMEM((TW,), jnp.float32),
        pltpu.VMEM((L,), jnp.float32),
        [pltpu.SemaphoreType.DMA for _ in range(NRING)],
    ],
)(_tl_body)


def _pack_bf16(embeddings):
    x = lax.bitcast_convert_type(embeddings, jnp.int32)
    lo, hi = x[:, :DP], x[:, DP:]
    lo_r = (lo + 0x7FFF + ((lo >> 16) & 1)) >> 16
    hi_r = hi + 0x7FFF + ((hi >> 16) & 1)
    return (hi_r & jnp.int32(-65536)) | (lo_r & 0xFFFF)


def kernel(embeddings, target, triplets):
    del target
    emb_pack = _pack_bf16(embeddings)
    aidx = triplets[:, 0].reshape(IDX_ROWS, CH)
    pidx = triplets[:, 1].reshape(IDX_ROWS, CH)
    nidx = triplets[:, 2].reshape(IDX_ROWS, CH)
    out_ap, out_an, out_td, out_part = _tl_kernel(emb_pack, aidx, pidx, nidx)
    loss = jnp.sum(out_part) / B
    tt = jnp.concatenate(
        [jnp.ones((B,), jnp.float32), jnp.zeros((B,), jnp.float32)])
    return loss, out_ap, out_an, out_td, tt
